# Initial kernel scaffold; baseline (speedup 1.0000x reference)
import jax
import jax.numpy as jnp
from jax.experimental import pallas as pl


def kernel(inputs, embeddings):
    raise NotImplementedError("write your pallas kernel here")


# speedup vs baseline: 1.0794x; 1.0794x over previous
import functools

import jax
import jax.numpy as jnp
import numpy as np
from jax import lax
from jax.experimental import pallas as pl
from jax.experimental.pallas import tpu as pltpu
from jax.experimental.pallas import tpu_sc as plsc


_BIG_I32 = np.int32(2**30)


def _argmin_body(k_chunk, n_chunks, x_ref, l2x_ref, e_ref, l2e_ref, codes_ref):
    m_t = x_ref.shape[0]
    x = x_ref[...]
    l2x = l2x_ref[0]

    def step(k, carry):
        best_val, best_idx = carry
        e = e_ref[pl.ds(k * k_chunk, k_chunk), :]
        l2e = l2e_ref[pl.ds(k * k_chunk, k_chunk), :]
        dot = lax.dot_general(e, x, (((1,), (1,)), ((), ())),
                              preferred_element_type=jnp.float32)
        dist = (l2x + l2e) - 2.0 * dot
        cmin = jnp.min(dist, axis=0, keepdims=True)
        iota = lax.broadcasted_iota(jnp.int32, (k_chunk, m_t), 0) + k * k_chunk
        cidx = jnp.min(jnp.where(dist == cmin, iota, _BIG_I32),
                       axis=0, keepdims=True)
        better = cmin < best_val
        return (jnp.where(better, cmin, best_val),
                jnp.where(better, cidx, best_idx))

    bv0 = jnp.full((1, m_t), jnp.inf, jnp.float32)
    bi0 = jnp.zeros((1, m_t), jnp.int32)
    _, best_idx = lax.fori_loop(0, n_chunks, step, (bv0, bi0))
    codes_ref[...] = best_idx[None]


def _codes_tc(x, l2x3, emb, l2e2, m_t=512, k_chunk=512):
    m, d = x.shape
    k, _ = emb.shape
    n_tiles = m // m_t
    n_chunks = k // k_chunk
    body = functools.partial(_argmin_body, k_chunk, n_chunks)
    return pl.pallas_call(
        body,
        grid=(n_tiles,),
        in_specs=[
            pl.BlockSpec((m_t, d), lambda i: (i, 0)),
            pl.BlockSpec((1, 1, m_t), lambda i: (i, 0, 0)),
            pl.BlockSpec((k, d), lambda i: (0, 0)),
            pl.BlockSpec((k, 1), lambda i: (0, 0)),
        ],
        out_specs=pl.BlockSpec((1, 1, m_t), lambda i: (i, 0, 0)),
        out_shape=jax.ShapeDtypeStruct((n_tiles, 1, m_t), jnp.int32),
    )(x, l2x3, emb, l2e2)


def _gather_sc(emb, codes_flat):
    k, d = emb.shape
    b = codes_flat.shape[0]
    info = plsc.get_sparse_core_info()
    nw = info.num_cores * info.num_subcores
    bpw = b // nw
    chunk = 128
    n_chunks = bpw // chunk
    mesh = plsc.VectorSubcoreMesh(core_axis_name="c", subcore_axis_name="s")

    @functools.partial(
        pl.kernel, mesh=mesh,
        out_type=jax.ShapeDtypeStruct((b, d), jnp.float32),
        scratch_types=[
            pltpu.VMEM((chunk,), jnp.int32),
            pltpu.VMEM((chunk, d), jnp.float32),
            pltpu.SemaphoreType.DMA,
        ],
    )
    def gather(emb_hbm, codes_hbm, out_hbm, idx_v, rows_v, sem):
        wid = lax.axis_index("s") * info.num_cores + lax.axis_index("c")
        base = wid * bpw
        for c in range(n_chunks):
            off = base + c * chunk
            pltpu.sync_copy(codes_hbm.at[pl.ds(off, chunk)], idx_v)
            pltpu.async_copy(emb_hbm.at[idx_v], rows_v, sem).wait()
            pltpu.sync_copy(rows_v, out_hbm.at[pl.ds(off, chunk)])

    return gather(emb, codes_flat)


def kernel(inputs, embeddings):
    b, h, w, d = inputs.shape
    m = b * h * w
    x = inputs.reshape(m, d)
    l2x = jnp.sum(inputs ** 2, axis=-1, keepdims=True)
    l2e = jnp.sum(embeddings ** 2, axis=-1)

    m_t = 512
    codes3 = _codes_tc(x, l2x.reshape(m // m_t, 1, m_t), embeddings,
                       l2e.reshape(-1, 1), m_t=m_t)
    codes_flat = codes3.reshape(m)
    code_vecs = _gather_sc(embeddings, codes_flat)
    return codes_flat.reshape(b, h, w), code_vecs.reshape(b, h, w, d)


# speedup vs baseline: 1.1130x; 1.0311x over previous
import functools

import jax
import jax.numpy as jnp
import numpy as np
from jax import lax
from jax.experimental import pallas as pl
from jax.experimental.pallas import tpu as pltpu
from jax.experimental.pallas import tpu_sc as plsc


_BIG_I32 = np.int32(2**30)


def _argmin_body(k_chunk, n_chunks, x_ref, l2x_ref, e2_ref, l2e_ref, codes_ref):
    m_t = x_ref.shape[0]
    x = x_ref[...]
    l2x = l2x_ref[0]
    iota_f = lax.broadcasted_iota(jnp.int32, (k_chunk, m_t), 0).astype(jnp.float32)

    def step(k, carry):
        best_val, best_idx = carry
        e2 = e2_ref[pl.ds(k * k_chunk, k_chunk), :]
        l2e = l2e_ref[pl.ds(k * k_chunk, k_chunk), :]
        dot2 = lax.dot_general(e2, x, (((1,), (1,)), ((), ())),
                               preferred_element_type=jnp.float32)
        dist = (l2x + l2e) - dot2
        cmin = jnp.min(dist, axis=0, keepdims=True)
        cidx = jnp.min(jnp.where(dist == cmin, iota_f, np.float32(65536.0)),
                       axis=0, keepdims=True)
        better = cmin < best_val
        gidx = cidx + (k * k_chunk).astype(jnp.float32)
        return (jnp.where(better, cmin, best_val),
                jnp.where(better, gidx, best_idx))

    bv0 = jnp.full((1, m_t), jnp.inf, jnp.float32)
    bi0 = jnp.zeros((1, m_t), jnp.float32)
    _, best_idx = lax.fori_loop(0, n_chunks, step, (bv0, bi0))
    codes_ref[...] = best_idx.astype(jnp.int32)[None]


def _codes_tc(x, l2x3, emb2, l2e2, m_t=512, k_chunk=512):
    m, d = x.shape
    k, _ = emb2.shape
    n_tiles = m // m_t
    n_chunks = k // k_chunk
    body = functools.partial(_argmin_body, k_chunk, n_chunks)
    return pl.pallas_call(
        body,
        grid=(n_tiles,),
        in_specs=[
            pl.BlockSpec((m_t, d), lambda i: (i, 0)),
            pl.BlockSpec((1, 1, m_t), lambda i: (i, 0, 0)),
            pl.BlockSpec((k, d), lambda i: (0, 0)),
            pl.BlockSpec((k, 1), lambda i: (0, 0)),
        ],
        out_specs=pl.BlockSpec((1, 1, m_t), lambda i: (i, 0, 0)),
        out_shape=jax.ShapeDtypeStruct((n_tiles, 1, m_t), jnp.int32),
    )(x, l2x3, emb2, l2e2)


def _gather_sc(emb, codes_flat):
    k, d = emb.shape
    b = codes_flat.shape[0]
    info = plsc.get_sparse_core_info()
    nw = info.num_cores * info.num_subcores
    bpw = b // nw
    chunk = 128
    n_chunks = bpw // chunk
    mesh = plsc.VectorSubcoreMesh(core_axis_name="c", subcore_axis_name="s")

    @functools.partial(
        pl.kernel, mesh=mesh,
        out_type=jax.ShapeDtypeStruct((b, d), jnp.float32),
        scratch_types=[
            pltpu.VMEM((chunk,), jnp.int32),
            pltpu.VMEM((chunk, d), jnp.float32),
            pltpu.SemaphoreType.DMA,
        ],
    )
    def gather(emb_hbm, codes_hbm, out_hbm, idx_v, rows_v, sem):
        wid = lax.axis_index("s") * info.num_cores + lax.axis_index("c")
        base = wid * bpw
        for c in range(n_chunks):
            off = base + c * chunk
            pltpu.sync_copy(codes_hbm.at[pl.ds(off, chunk)], idx_v)
            pltpu.async_copy(emb_hbm.at[idx_v], rows_v, sem).wait()
            pltpu.sync_copy(rows_v, out_hbm.at[pl.ds(off, chunk)])

    return gather(emb, codes_flat)


def kernel(inputs, embeddings):
    b, h, w, d = inputs.shape
    m = b * h * w
    x = inputs.reshape(m, d)
    l2x = jnp.sum(inputs ** 2, axis=-1, keepdims=True)
    l2e = jnp.sum(embeddings ** 2, axis=-1)

    m_t = 512
    codes3 = _codes_tc(x, l2x.reshape(m // m_t, 1, m_t),
                       embeddings + embeddings, l2e.reshape(-1, 1), m_t=m_t)
    codes_flat = codes3.reshape(m)
    code_vecs = _gather_sc(embeddings, codes_flat)
    return codes_flat.reshape(b, h, w), code_vecs.reshape(b, h, w, d)


# speedup vs baseline: 1.8069x; 1.6235x over previous
import functools

import jax
import jax.numpy as jnp
import numpy as np
from jax import lax
from jax.experimental import pallas as pl
from jax.experimental.pallas import tpu as pltpu
from jax.experimental.pallas import tpu_sc as plsc


_BIG_I32 = np.int32(2**30)


_SL = 16


def _argmin_body(k_chunk, n_chunks, x_ref, l2x_ref, e2_ref, l2e_ref, codes_ref):
    m_t = x_ref.shape[0]
    sl = _SL
    n_sl = k_chunk // sl
    x = x_ref[...]
    l2x = l2x_ref[0]
    iota_sub = lax.broadcasted_iota(jnp.int32, (sl, m_t), 0).astype(jnp.float32)

    def step(k, carry):
        rmin, rix = carry
        e2 = e2_ref[pl.ds(k * k_chunk, k_chunk), :]
        l2e = l2e_ref[pl.ds(k * k_chunk, k_chunk), :]
        dot2 = lax.dot_general(e2, x, (((1,), (1,)), ((), ())),
                               preferred_element_type=jnp.float32)
        kf = (k * n_sl).astype(jnp.float32)
        for s in range(n_sl):
            d = lax.slice(dot2, (s * sl, 0), ((s + 1) * sl, m_t))
            l2e_s = lax.slice(l2e, (s * sl, 0), ((s + 1) * sl, 1))
            dist = (l2x + l2e_s) - d
            mask = dist < rmin
            rmin = jnp.minimum(rmin, dist)
            rix = jnp.where(mask, kf + np.float32(s), rix)
        return rmin, rix

    rmin0 = jnp.full((sl, m_t), jnp.inf, jnp.float32)
    rix0 = jnp.zeros((sl, m_t), jnp.float32)
    rmin, rix = lax.fori_loop(0, n_chunks, step, (rmin0, rix0))
    gmin = jnp.min(rmin, axis=0, keepdims=True)
    gidx = rix * np.float32(sl) + iota_sub
    cand = jnp.where(rmin == gmin, gidx, np.float32(65536.0))
    best = jnp.min(cand, axis=0, keepdims=True)
    codes_ref[...] = best.astype(jnp.int32)[None]


def _codes_tc(x, l2x3, emb2, l2e2, m_t=512, k_chunk=1024):
    m, d = x.shape
    k, _ = emb2.shape
    n_tiles = m // m_t
    n_chunks = k // k_chunk
    body = functools.partial(_argmin_body, k_chunk, n_chunks)
    return pl.pallas_call(
        body,
        grid=(n_tiles,),
        in_specs=[
            pl.BlockSpec((m_t, d), lambda i: (i, 0)),
            pl.BlockSpec((1, 1, m_t), lambda i: (i, 0, 0)),
            pl.BlockSpec((k, d), lambda i: (0, 0)),
            pl.BlockSpec((k, 1), lambda i: (0, 0)),
        ],
        out_specs=pl.BlockSpec((1, 1, m_t), lambda i: (i, 0, 0)),
        out_shape=jax.ShapeDtypeStruct((n_tiles, 1, m_t), jnp.int32),
    )(x, l2x3, emb2, l2e2)


def _gather_sc(emb, codes_flat):
    k, d = emb.shape
    b = codes_flat.shape[0]
    info = plsc.get_sparse_core_info()
    nw = info.num_cores * info.num_subcores
    bpw = b // nw
    chunk = 128
    n_chunks = bpw // chunk
    mesh = plsc.VectorSubcoreMesh(core_axis_name="c", subcore_axis_name="s")

    @functools.partial(
        pl.kernel, mesh=mesh,
        out_type=jax.ShapeDtypeStruct((b, d), jnp.float32),
        scratch_types=[
            pltpu.VMEM((chunk,), jnp.int32),
            pltpu.VMEM((chunk, d), jnp.float32),
            pltpu.SemaphoreType.DMA,
        ],
    )
    def gather(emb_hbm, codes_hbm, out_hbm, idx_v, rows_v, sem):
        wid = lax.axis_index("s") * info.num_cores + lax.axis_index("c")
        base = wid * bpw
        for c in range(n_chunks):
            off = base + c * chunk
            pltpu.sync_copy(codes_hbm.at[pl.ds(off, chunk)], idx_v)
            pltpu.async_copy(emb_hbm.at[idx_v], rows_v, sem).wait()
            pltpu.sync_copy(rows_v, out_hbm.at[pl.ds(off, chunk)])

    return gather(emb, codes_flat)


def kernel(inputs, embeddings):
    b, h, w, d = inputs.shape
    m = b * h * w
    x = inputs.reshape(m, d)
    l2x = jnp.sum(inputs ** 2, axis=-1, keepdims=True)
    l2e = jnp.sum(embeddings ** 2, axis=-1)

    m_t = 512
    codes3 = _codes_tc(x, l2x.reshape(m // m_t, 1, m_t),
                       embeddings + embeddings, l2e.reshape(-1, 1), m_t=m_t)
    codes_flat = codes3.reshape(m)
    code_vecs = _gather_sc(embeddings, codes_flat)
    return codes_flat.reshape(b, h, w), code_vecs.reshape(b, h, w, d)


# speedup vs baseline: 2.4192x; 1.3388x over previous
import functools

import jax
import jax.numpy as jnp
import numpy as np
from jax import lax
from jax.experimental import pallas as pl
from jax.experimental.pallas import tpu as pltpu
from jax.experimental.pallas import tpu_sc as plsc


_BIG_I32 = np.int32(2**30)


_SL = 16


def _argmin_body(k_chunk, n_chunks, x_ref, l2x_ref, e2_ref, l2e_ref, codes_ref):
    m_t = x_ref.shape[0]
    sl = _SL
    n_sl = k_chunk // sl
    x = x_ref[...]
    l2x = l2x_ref[0]
    iota_sub = lax.broadcasted_iota(jnp.int32, (sl, m_t), 0).astype(jnp.float32)

    rmin = jnp.full((sl, m_t), jnp.inf, jnp.float32)
    rix = jnp.zeros((sl, m_t), jnp.float32)
    for k in range(n_chunks):
        e2 = e2_ref[pl.ds(k * k_chunk, k_chunk), :]
        l2e = l2e_ref[pl.ds(k * k_chunk, k_chunk), :]
        dot2 = lax.dot_general(e2, x, (((1,), (1,)), ((), ())),
                               preferred_element_type=jnp.float32)
        for s in range(n_sl):
            d = lax.slice(dot2, (s * sl, 0), ((s + 1) * sl, m_t))
            l2e_s = lax.slice(l2e, (s * sl, 0), ((s + 1) * sl, 1))
            dist = (l2x + l2e_s) - d
            mask = dist < rmin
            rmin = jnp.minimum(rmin, dist)
            rix = jnp.where(mask, np.float32(k * n_sl + s), rix)
    gmin = jnp.min(rmin, axis=0, keepdims=True)
    gidx = rix * np.float32(sl) + iota_sub
    cand = jnp.where(rmin == gmin, gidx, np.float32(65536.0))
    best = jnp.min(cand, axis=0, keepdims=True)
    codes_ref[...] = best.astype(jnp.int32)[None]


def _codes_tc(x, l2x3, emb2, l2e2, m_t=512, k_chunk=1024):
    m, d = x.shape
    k, _ = emb2.shape
    n_tiles = m // m_t
    n_chunks = k // k_chunk
    body = functools.partial(_argmin_body, k_chunk, n_chunks)
    return pl.pallas_call(
        body,
        grid=(n_tiles,),
        in_specs=[
            pl.BlockSpec((m_t, d), lambda i: (i, 0)),
            pl.BlockSpec((1, 1, m_t), lambda i: (i, 0, 0)),
            pl.BlockSpec((k, d), lambda i: (0, 0)),
            pl.BlockSpec((k, 1), lambda i: (0, 0)),
        ],
        out_specs=pl.BlockSpec((1, 1, m_t), lambda i: (i, 0, 0)),
        out_shape=jax.ShapeDtypeStruct((n_tiles, 1, m_t), jnp.int32),
    )(x, l2x3, emb2, l2e2)


def _gather_sc(emb, codes_flat):
    k, d = emb.shape
    b = codes_flat.shape[0]
    info = plsc.get_sparse_core_info()
    nw = info.num_cores * info.num_subcores
    bpw = b // nw
    chunk = 128
    n_chunks = bpw // chunk
    mesh = plsc.VectorSubcoreMesh(core_axis_name="c", subcore_axis_name="s")

    @functools.partial(
        pl.kernel, mesh=mesh,
        out_type=jax.ShapeDtypeStruct((b, d), jnp.float32),
        scratch_types=[
            pltpu.VMEM((chunk,), jnp.int32),
            pltpu.VMEM((chunk, d), jnp.float32),
            pltpu.SemaphoreType.DMA,
        ],
    )
    def gather(emb_hbm, codes_hbm, out_hbm, idx_v, rows_v, sem):
        wid = lax.axis_index("s") * info.num_cores + lax.axis_index("c")
        base = wid * bpw
        for c in range(n_chunks):
            off = base + c * chunk
            pltpu.sync_copy(codes_hbm.at[pl.ds(off, chunk)], idx_v)
            pltpu.async_copy(emb_hbm.at[idx_v], rows_v, sem).wait()
            pltpu.sync_copy(rows_v, out_hbm.at[pl.ds(off, chunk)])

    return gather(emb, codes_flat)


def kernel(inputs, embeddings):
    b, h, w, d = inputs.shape
    m = b * h * w
    x = inputs.reshape(m, d)
    l2x = jnp.sum(inputs ** 2, axis=-1, keepdims=True)
    l2e = jnp.sum(embeddings ** 2, axis=-1)

    m_t = 512
    codes3 = _codes_tc(x, l2x.reshape(m // m_t, 1, m_t),
                       embeddings + embeddings, l2e.reshape(-1, 1), m_t=m_t)
    codes_flat = codes3.reshape(m)
    code_vecs = _gather_sc(embeddings, codes_flat)
    return codes_flat.reshape(b, h, w), code_vecs.reshape(b, h, w, d)


# speedup vs baseline: 2.4856x; 1.0275x over previous
import functools

import jax
import jax.numpy as jnp
import numpy as np
from jax import lax
from jax.experimental import pallas as pl
from jax.experimental.pallas import tpu as pltpu
from jax.experimental.pallas import tpu_sc as plsc


_BIG_I32 = np.int32(2**30)


_SL = 16


def _argmin_body(k_chunk, n_chunks, x_ref, l2x_ref, e2_ref, l2e_ref, codes_ref):
    m_t = x_ref.shape[0]
    sl = _SL
    n_sl = k_chunk // sl
    x = x_ref[...]
    l2x = l2x_ref[0]
    iota_sub = lax.broadcasted_iota(jnp.int32, (sl, m_t), 0).astype(jnp.float32)

    rmin = jnp.full((sl, m_t), jnp.inf, jnp.float32)
    rix = jnp.zeros((sl, m_t), jnp.float32)
    for k in range(n_chunks):
        e2 = e2_ref[pl.ds(k * k_chunk, k_chunk), :]
        l2e = l2e_ref[pl.ds(k * k_chunk, k_chunk), :]
        dot2 = lax.dot_general(e2, x, (((1,), (1,)), ((), ())),
                               preferred_element_type=jnp.float32)
        for s in range(n_sl):
            d = lax.slice(dot2, (s * sl, 0), ((s + 1) * sl, m_t))
            l2e_s = lax.slice(l2e, (s * sl, 0), ((s + 1) * sl, 1))
            dist = (l2x + l2e_s) - d
            mask = dist < rmin
            rmin = jnp.minimum(rmin, dist)
            rix = jnp.where(mask, np.float32(k * n_sl + s), rix)
    gmin = jnp.min(rmin, axis=0, keepdims=True)
    gidx = rix * np.float32(sl) + iota_sub
    cand = jnp.where(rmin == gmin, gidx, np.float32(65536.0))
    best = jnp.min(cand, axis=0, keepdims=True)
    codes_ref[...] = best.astype(jnp.int32)[None]


def _codes_tc(x, l2x3, emb2, l2e2, m_t=512, k_chunk=1024):
    m, d = x.shape
    k, _ = emb2.shape
    n_tiles = m // m_t
    n_chunks = k // k_chunk
    body = functools.partial(_argmin_body, k_chunk, n_chunks)
    return pl.pallas_call(
        body,
        grid=(n_tiles,),
        in_specs=[
            pl.BlockSpec((m_t, d), lambda i: (i, 0)),
            pl.BlockSpec((1, 1, m_t), lambda i: (i, 0, 0)),
            pl.BlockSpec((k, d), lambda i: (0, 0)),
            pl.BlockSpec((k, 1), lambda i: (0, 0)),
        ],
        out_specs=pl.BlockSpec((1, 1, m_t), lambda i: (i, 0, 0)),
        out_shape=jax.ShapeDtypeStruct((n_tiles, 1, m_t), jnp.int32),
    )(x, l2x3, emb2, l2e2)


def _gather_sc(emb, codes_flat):
    k, d = emb.shape
    b = codes_flat.shape[0]
    info = plsc.get_sparse_core_info()
    nw = info.num_cores * info.num_subcores
    bpw = b // nw
    chunk = 128
    n_chunks = bpw // chunk
    mesh = plsc.VectorSubcoreMesh(core_axis_name="c", subcore_axis_name="s")

    @functools.partial(
        pl.kernel, mesh=mesh,
        out_type=jax.ShapeDtypeStruct((b, d), jnp.float32),
        scratch_types=[
            pltpu.VMEM((chunk,), jnp.int32),
            pltpu.VMEM((chunk, d), jnp.float32),
            pltpu.SemaphoreType.DMA,
        ],
    )
    def gather(emb_hbm, codes_hbm, out_hbm, idx_v, rows_v, sem):
        wid = lax.axis_index("s") * info.num_cores + lax.axis_index("c")
        base = wid * bpw
        for c in range(n_chunks):
            off = base + c * chunk
            pltpu.sync_copy(codes_hbm.at[pl.ds(off, chunk)], idx_v)
            pltpu.async_copy(emb_hbm.at[idx_v], rows_v, sem).wait()
            pltpu.sync_copy(rows_v, out_hbm.at[pl.ds(off, chunk)])

    return gather(emb, codes_flat)


def kernel(inputs, embeddings):
    b, h, w, d = inputs.shape
    m = b * h * w
    x = inputs.reshape(m, d)
    l2x = jnp.sum(inputs ** 2, axis=-1, keepdims=True)
    l2e = jnp.sum(embeddings ** 2, axis=-1)

    m_t = 1024
    codes3 = _codes_tc(x, l2x.reshape(m // m_t, 1, m_t),
                       embeddings + embeddings, l2e.reshape(-1, 1), m_t=m_t)
    codes_flat = codes3.reshape(m)
    code_vecs = _gather_sc(embeddings, codes_flat)
    return codes_flat.reshape(b, h, w), code_vecs.reshape(b, h, w, d)


# speedup vs baseline: 2.5561x; 1.0284x over previous
import functools

import jax
import jax.numpy as jnp
import numpy as np
from jax import lax
from jax.experimental import pallas as pl
from jax.experimental.pallas import tpu as pltpu
from jax.experimental.pallas import tpu_sc as plsc


_BIG_I32 = np.int32(2**30)


_SL = 16


def _argmin_body(k_chunk, n_chunks, x_ref, l2x_ref, e2_ref, l2e_ref, codes_ref):
    m_t = x_ref.shape[0]
    sl = _SL
    n_sl = k_chunk // sl
    x = x_ref[...]
    l2x = l2x_ref[0]
    iota_sub = lax.broadcasted_iota(jnp.int32, (sl, m_t), 0).astype(jnp.float32)

    rmin = jnp.full((sl, m_t), jnp.inf, jnp.float32)
    rix = jnp.zeros((sl, m_t), jnp.float32)
    for k in range(n_chunks):
        e2 = e2_ref[pl.ds(k * k_chunk, k_chunk), :]
        l2e = l2e_ref[pl.ds(k * k_chunk, k_chunk), :]
        dot2 = lax.dot_general(e2, x, (((1,), (1,)), ((), ())),
                               preferred_element_type=jnp.float32)
        for s in range(n_sl):
            d = lax.slice(dot2, (s * sl, 0), ((s + 1) * sl, m_t))
            l2e_s = lax.slice(l2e, (s * sl, 0), ((s + 1) * sl, 1))
            dist = (l2x + l2e_s) - d
            mask = dist < rmin
            rmin = jnp.minimum(rmin, dist)
            rix = jnp.where(mask, np.float32(k * n_sl + s), rix)
    gmin = jnp.min(rmin, axis=0, keepdims=True)
    gidx = rix * np.float32(sl) + iota_sub
    cand = jnp.where(rmin == gmin, gidx, np.float32(65536.0))
    best = jnp.min(cand, axis=0, keepdims=True)
    codes_ref[...] = best.astype(jnp.int32)[None]


def _codes_tc(x, l2x3, emb2, l2e2, m_t=512, k_chunk=2048):
    m, d = x.shape
    k, _ = emb2.shape
    n_tiles = m // m_t
    n_chunks = k // k_chunk
    body = functools.partial(_argmin_body, k_chunk, n_chunks)
    return pl.pallas_call(
        body,
        grid=(n_tiles,),
        in_specs=[
            pl.BlockSpec((m_t, d), lambda i: (i, 0)),
            pl.BlockSpec((1, 1, m_t), lambda i: (i, 0, 0)),
            pl.BlockSpec((k, d), lambda i: (0, 0)),
            pl.BlockSpec((k, 1), lambda i: (0, 0)),
        ],
        out_specs=pl.BlockSpec((1, 1, m_t), lambda i: (i, 0, 0)),
        out_shape=jax.ShapeDtypeStruct((n_tiles, 1, m_t), jnp.int32),
    )(x, l2x3, emb2, l2e2)


def _gather_sc(emb, codes_flat):
    k, d = emb.shape
    b = codes_flat.shape[0]
    info = plsc.get_sparse_core_info()
    nw = info.num_cores * info.num_subcores
    bpw = b // nw
    chunk = 128
    n_chunks = bpw // chunk
    mesh = plsc.VectorSubcoreMesh(core_axis_name="c", subcore_axis_name="s")

    @functools.partial(
        pl.kernel, mesh=mesh,
        out_type=jax.ShapeDtypeStruct((b, d), jnp.float32),
        scratch_types=[
            pltpu.VMEM((chunk,), jnp.int32),
            pltpu.VMEM((chunk, d), jnp.float32),
            pltpu.SemaphoreType.DMA,
        ],
    )
    def gather(emb_hbm, codes_hbm, out_hbm, idx_v, rows_v, sem):
        wid = lax.axis_index("s") * info.num_cores + lax.axis_index("c")
        base = wid * bpw
        for c in range(n_chunks):
            off = base + c * chunk
            pltpu.sync_copy(codes_hbm.at[pl.ds(off, chunk)], idx_v)
            pltpu.async_copy(emb_hbm.at[idx_v], rows_v, sem).wait()
            pltpu.sync_copy(rows_v, out_hbm.at[pl.ds(off, chunk)])

    return gather(emb, codes_flat)


def kernel(inputs, embeddings):
    b, h, w, d = inputs.shape
    m = b * h * w
    x = inputs.reshape(m, d)
    l2x = jnp.sum(inputs ** 2, axis=-1, keepdims=True)
    l2e = jnp.sum(embeddings ** 2, axis=-1)

    m_t = 1024
    codes3 = _codes_tc(x, l2x.reshape(m // m_t, 1, m_t),
                       embeddings + embeddings, l2e.reshape(-1, 1), m_t=m_t)
    codes_flat = codes3.reshape(m)
    code_vecs = _gather_sc(embeddings, codes_flat)
    return codes_flat.reshape(b, h, w), code_vecs.reshape(b, h, w, d)


# speedup vs baseline: 2.5831x; 1.0105x over previous
import functools

import jax
import jax.numpy as jnp
import numpy as np
from jax import lax
from jax.experimental import pallas as pl
from jax.experimental.pallas import tpu as pltpu
from jax.experimental.pallas import tpu_sc as plsc


_BIG_I32 = np.int32(2**30)


_SL = 16


def _argmin_body(k_chunk, n_chunks, x_ref, l2x_ref, e2_ref, l2e_ref, codes_ref):
    m_t = x_ref.shape[0]
    sl = _SL
    n_sl = k_chunk // sl
    x = x_ref[...]
    l2x = l2x_ref[0]
    iota_sub = lax.broadcasted_iota(jnp.int32, (sl, m_t), 0).astype(jnp.float32)

    rmin = jnp.full((sl, m_t), jnp.inf, jnp.float32)
    rix = jnp.zeros((sl, m_t), jnp.float32)
    for k in range(n_chunks):
        e2 = e2_ref[pl.ds(k * k_chunk, k_chunk), :]
        l2e = l2e_ref[pl.ds(k * k_chunk, k_chunk), :]
        dot2 = lax.dot_general(e2, x, (((1,), (1,)), ((), ())),
                               preferred_element_type=jnp.float32)
        for s in range(n_sl):
            d = lax.slice(dot2, (s * sl, 0), ((s + 1) * sl, m_t))
            l2e_s = lax.slice(l2e, (s * sl, 0), ((s + 1) * sl, 1))
            dist = (l2x + l2e_s) - d
            mask = dist < rmin
            rmin = jnp.minimum(rmin, dist)
            rix = jnp.where(mask, np.float32(k * n_sl + s), rix)
    gmin = jnp.min(rmin, axis=0, keepdims=True)
    gidx = rix * np.float32(sl) + iota_sub
    cand = jnp.where(rmin == gmin, gidx, np.float32(65536.0))
    best = jnp.min(cand, axis=0, keepdims=True)
    codes_ref[...] = best.astype(jnp.int32)[None]


def _codes_tc(x, l2x3, emb2, l2e2, m_t=512, k_chunk=2048):
    m, d = x.shape
    k, _ = emb2.shape
    n_tiles = m // m_t
    n_chunks = k // k_chunk
    body = functools.partial(_argmin_body, k_chunk, n_chunks)
    return pl.pallas_call(
        body,
        grid=(n_tiles,),
        in_specs=[
            pl.BlockSpec((m_t, d), lambda i: (i, 0)),
            pl.BlockSpec((1, 1, m_t), lambda i: (i, 0, 0)),
            pl.BlockSpec((k, d), lambda i: (0, 0)),
            pl.BlockSpec((k, 1), lambda i: (0, 0)),
        ],
        out_specs=pl.BlockSpec((1, 1, m_t), lambda i: (i, 0, 0)),
        out_shape=jax.ShapeDtypeStruct((n_tiles, 1, m_t), jnp.int32),
    )(x, l2x3, emb2, l2e2)


def _gather_sc(emb, codes_flat):
    k, d = emb.shape
    b = codes_flat.shape[0]
    info = plsc.get_sparse_core_info()
    nw = info.num_cores * info.num_subcores
    bpw = b // nw
    chunk = 128
    n_chunks = bpw // chunk
    mesh = plsc.VectorSubcoreMesh(core_axis_name="c", subcore_axis_name="s")

    @functools.partial(
        pl.kernel, mesh=mesh,
        out_type=jax.ShapeDtypeStruct((b, d), jnp.float32),
        scratch_types=[
            pltpu.VMEM((chunk,), jnp.int32),
            pltpu.VMEM((chunk, d), jnp.float32),
            pltpu.SemaphoreType.DMA,
        ],
    )
    def gather(emb_hbm, codes_hbm, out_hbm, idx_v, rows_v, sem):
        wid = lax.axis_index("s") * info.num_cores + lax.axis_index("c")
        base = wid * bpw
        for c in range(n_chunks):
            off = base + c * chunk
            pltpu.sync_copy(codes_hbm.at[pl.ds(off, chunk)], idx_v)
            pltpu.async_copy(emb_hbm.at[idx_v], rows_v, sem).wait()
            pltpu.sync_copy(rows_v, out_hbm.at[pl.ds(off, chunk)])

    return gather(emb, codes_flat)


def kernel(inputs, embeddings):
    b, h, w, d = inputs.shape
    m = b * h * w
    x = inputs.reshape(m, d)
    l2x = jnp.sum(inputs ** 2, axis=-1, keepdims=True)
    l2e = jnp.sum(embeddings ** 2, axis=-1)

    m_t = 2048
    codes3 = _codes_tc(x, l2x.reshape(m // m_t, 1, m_t),
                       embeddings + embeddings, l2e.reshape(-1, 1), m_t=m_t)
    codes_flat = codes3.reshape(m)
    code_vecs = _gather_sc(embeddings, codes_flat)
    return codes_flat.reshape(b, h, w), code_vecs.reshape(b, h, w, d)


# speedup vs baseline: 2.6351x; 1.0202x over previous
import functools

import jax
import jax.numpy as jnp
import numpy as np
from jax import lax
from jax.experimental import pallas as pl
from jax.experimental.pallas import tpu as pltpu
from jax.experimental.pallas import tpu_sc as plsc


_BIG_I32 = np.int32(2**30)


_SL = 16


def _argmin_body(k_chunk, n_chunks, x_ref, l2x_ref, e2_ref, l2e_ref, codes_ref):
    m_t = x_ref.shape[0]
    sl = _SL
    n_sl = k_chunk // sl
    x = x_ref[...]
    l2x = l2x_ref[0]
    iota_sub = lax.broadcasted_iota(jnp.int32, (sl, m_t), 0).astype(jnp.float32)

    rmin = jnp.full((sl, m_t), jnp.inf, jnp.float32)
    rix = jnp.zeros((sl, m_t), jnp.float32)
    for k in range(n_chunks):
        e2 = e2_ref[pl.ds(k * k_chunk, k_chunk), :]
        l2e = l2e_ref[pl.ds(k * k_chunk, k_chunk), :]
        dot2 = lax.dot_general(e2, x, (((1,), (1,)), ((), ())),
                               preferred_element_type=jnp.float32)
        for s in range(n_sl):
            d = lax.slice(dot2, (s * sl, 0), ((s + 1) * sl, m_t))
            l2e_s = lax.slice(l2e, (s * sl, 0), ((s + 1) * sl, 1))
            dist = (l2x + l2e_s) - d
            mask = dist < rmin
            rmin = jnp.minimum(rmin, dist)
            rix = jnp.where(mask, np.float32(k * n_sl + s), rix)
    gmin = jnp.min(rmin, axis=0, keepdims=True)
    gidx = rix * np.float32(sl) + iota_sub
    cand = jnp.where(rmin == gmin, gidx, np.float32(65536.0))
    best = jnp.min(cand, axis=0, keepdims=True)
    codes_ref[...] = best.astype(jnp.int32)[None]


def _codes_tc(x, l2x3, emb2, l2e2, m_t=512, k_chunk=4096):
    m, d = x.shape
    k, _ = emb2.shape
    n_tiles = m // m_t
    n_chunks = k // k_chunk
    body = functools.partial(_argmin_body, k_chunk, n_chunks)
    return pl.pallas_call(
        body,
        grid=(n_tiles,),
        in_specs=[
            pl.BlockSpec((m_t, d), lambda i: (i, 0)),
            pl.BlockSpec((1, 1, m_t), lambda i: (i, 0, 0)),
            pl.BlockSpec((k, d), lambda i: (0, 0)),
            pl.BlockSpec((k, 1), lambda i: (0, 0)),
        ],
        out_specs=pl.BlockSpec((1, 1, m_t), lambda i: (i, 0, 0)),
        out_shape=jax.ShapeDtypeStruct((n_tiles, 1, m_t), jnp.int32),
    )(x, l2x3, emb2, l2e2)


def _gather_sc(emb, codes_flat):
    k, d = emb.shape
    b = codes_flat.shape[0]
    info = plsc.get_sparse_core_info()
    nw = info.num_cores * info.num_subcores
    bpw = b // nw
    chunk = 128
    n_chunks = bpw // chunk
    mesh = plsc.VectorSubcoreMesh(core_axis_name="c", subcore_axis_name="s")

    @functools.partial(
        pl.kernel, mesh=mesh,
        out_type=jax.ShapeDtypeStruct((b, d), jnp.float32),
        scratch_types=[
            pltpu.VMEM((chunk,), jnp.int32),
            pltpu.VMEM((chunk, d), jnp.float32),
            pltpu.SemaphoreType.DMA,
        ],
    )
    def gather(emb_hbm, codes_hbm, out_hbm, idx_v, rows_v, sem):
        wid = lax.axis_index("s") * info.num_cores + lax.axis_index("c")
        base = wid * bpw
        for c in range(n_chunks):
            off = base + c * chunk
            pltpu.sync_copy(codes_hbm.at[pl.ds(off, chunk)], idx_v)
            pltpu.async_copy(emb_hbm.at[idx_v], rows_v, sem).wait()
            pltpu.sync_copy(rows_v, out_hbm.at[pl.ds(off, chunk)])

    return gather(emb, codes_flat)


def kernel(inputs, embeddings):
    b, h, w, d = inputs.shape
    m = b * h * w
    x = inputs.reshape(m, d)
    l2x = jnp.sum(inputs ** 2, axis=-1, keepdims=True)
    l2e = jnp.sum(embeddings ** 2, axis=-1)

    m_t = 2048
    codes3 = _codes_tc(x, l2x.reshape(m // m_t, 1, m_t),
                       embeddings + embeddings, l2e.reshape(-1, 1), m_t=m_t)
    codes_flat = codes3.reshape(m)
    code_vecs = _gather_sc(embeddings, codes_flat)
    return codes_flat.reshape(b, h, w), code_vecs.reshape(b, h, w, d)
